# baseline (speedup 1.0000x reference)
import functools

import jax
import jax.numpy as jnp
from jax import lax
from jax.experimental import pallas as pl
from jax.experimental.pallas import tpu as pltpu
from jax.experimental.pallas import tpu_sc as plsc

VOCAB = 1000000
D = 32
B = 16384
NC = 2
NS = 16
NW = NC * NS
BPW = B // NW
CH = 128
NCH = BPW // CH

_mesh = plsc.VectorSubcoreMesh(core_axis_name="c", subcore_axis_name="s")


@functools.partial(
    pl.kernel,
    mesh=_mesh,
    out_type=[
        jax.ShapeDtypeStruct((B, D), jnp.float32),
        jax.ShapeDtypeStruct((16,), jnp.float32),
    ],
    scratch_types=[
        pltpu.VMEM((NCH, CH), jnp.int32),
        pltpu.VMEM((BPW, D), jnp.float32),
        pltpu.VMEM((16,), jnp.float32),
        pltpu.SemaphoreType.DMA,
    ],
    compiler_params=pltpu.CompilerParams(use_tc_tiling_on_sc=False),
)
def _policy_gather(idx_hbm, x_hbm, sig_hbm, out_hbm, sig_out_hbm,
                   idx_v, rows_v, sig_v, sem):
    wid = lax.axis_index("s") * NC + lax.axis_index("c")
    base = wid * BPW

    pltpu.sync_copy(idx_hbm.at[wid], idx_v)

    copies = [
        pltpu.async_copy(x_hbm.at[idx_v.at[j]],
                         rows_v.at[pl.ds(j * CH, CH)], sem)
        for j in range(NCH)
    ]
    for c in copies:
        c.wait()

    pltpu.sync_copy(rows_v, out_hbm.at[pl.ds(base, BPW)])

    @pl.when(wid == 0)
    def _clip_sigma():
        pltpu.sync_copy(sig_hbm, sig_v)
        v = sig_v[...]
        sig_v[...] = jnp.minimum(jnp.maximum(v, jnp.float32(-2.5)),
                                 jnp.float32(0.0))
        pltpu.sync_copy(sig_v, sig_out_hbm)


def kernel(indices, X, log_sigma):
    idx3 = indices.reshape(NW, NCH, CH)
    sig16 = jnp.broadcast_to(log_sigma, (16,))
    out, sig = _policy_gather(idx3, X, sig16)
    return out, sig[:1]
